# baseline (speedup 1.0000x reference)
import functools

import jax
import jax.numpy as jnp
from jax import lax
from jax.experimental import pallas as pl
from jax.experimental.pallas import tpu as pltpu
from jax.experimental.pallas import tpu_sc as plsc

VOCAB = 100000
D = 64
BATCH = 16384
HIST = 50

NC = 2
NS = 16
NW = NC * NS

PER_W = BATCH * HIST // NW
G = 2
CHUNK = G * HIST
NCH = PER_W // CHUNK
GRP = 8
RPG = GRP * CHUNK
NG = NCH // GRP

_mesh = plsc.VectorSubcoreMesh(core_axis_name="c", subcore_axis_name="s")


@functools.partial(
    pl.kernel,
    mesh=_mesh,
    out_type=jax.ShapeDtypeStruct((NW, HIST, D), jnp.float32),
    scratch_types=[
        pltpu.VMEM((NCH, CHUNK), jnp.int32),
        pltpu.VMEM((RPG, D), jnp.bfloat16),
        pltpu.VMEM((RPG, D), jnp.bfloat16),
        pltpu.VMEM((HIST, D), jnp.float32),
        pltpu.SemaphoreType.DMA,
        pltpu.SemaphoreType.DMA,
    ],
    compiler_params=pltpu.CompilerParams(
        use_tc_tiling_on_sc=False, needs_layout_passes=False
    ),
)
def _sc_partial_sums(idx_hbm, table_hbm, out_hbm, idxv, rows0, rows1, acc, sem0, sem1):
    wid = lax.axis_index("s") * NC + lax.axis_index("c")

    zero = jnp.zeros((16,), jnp.float32)

    def zbody(l, carry):
        for d in range(D // 16):
            acc[l, pl.ds(d * 16, 16)] = zero
        return carry

    lax.fori_loop(0, HIST, zbody, 0)

    pltpu.sync_copy(idx_hbm.at[wid], idxv)

    def fire(grp, rows, sem):
        for k in range(GRP):
            pltpu.async_copy(
                table_hbm.at[idxv.at[grp * GRP + k]],
                rows.at[pl.ds(k * CHUNK, CHUNK)],
                sem,
            )

    def drain(rows, sem):
        pltpu.make_async_copy(table_hbm.at[pl.ds(0, RPG)], rows, sem).wait()

    def accumulate(rows):
        def abody(l, inner):
            for dd in range(D // 32):
                sl = pl.ds(dd * 32, 32)
                s = rows[l, sl]
                for g in range(1, GRP * G):
                    s = s + rows[g * HIST + l, sl]
                even, odd = plsc.unpack(
                    s,
                    format=plsc.PackFormat.INTERLEAVED,
                    preferred_element_type=jnp.float32,
                )
                plsc.addupdate(acc.at[l, pl.ds(dd * 32, 16)], even)
                plsc.addupdate(acc.at[l, pl.ds(dd * 32 + 16, 16)], odd)
            return inner

        lax.fori_loop(0, HIST, abody, 0)

    fire(0, rows0, sem0)

    def group_body(i, carry):
        fire(2 * i + 1, rows1, sem1)
        drain(rows0, sem0)
        accumulate(rows0)

        @pl.when(2 * i + 2 < NG)
        def _():
            fire(2 * i + 2, rows0, sem0)

        drain(rows1, sem1)
        accumulate(rows1)
        return carry

    lax.fori_loop(0, NG // 2, group_body, 0)

    pltpu.sync_copy(acc, out_hbm.at[wid])


VC = 8192


def _mm_body(part_ref, w_ref, b_ref, o_ref):
    mean = jnp.sum(part_ref[...], axis=0) * (1.0 / BATCH)
    o_ref[...] = (
        lax.dot_general(
            mean, w_ref[...], (((1,), (1,)), ((), ())),
            preferred_element_type=jnp.float32,
        )
        + b_ref[...]
    )


_project = pl.pallas_call(
    _mm_body,
    grid=(pl.cdiv(VOCAB, VC),),
    in_specs=[
        pl.BlockSpec((NW, HIST, D), lambda j: (0, 0, 0)),
        pl.BlockSpec((VC, D), lambda j: (j, 0)),
        pl.BlockSpec((1, VC), lambda j: (0, j)),
    ],
    out_specs=pl.BlockSpec((HIST, VC), lambda j: (0, j)),
    out_shape=jax.ShapeDtypeStruct((HIST, VOCAB), jnp.float32),
)


def kernel(context_idxs, emb_table, W, b):
    idx = context_idxs.astype(jnp.int32).reshape(NW, NCH, CHUNK)
    table16 = emb_table.astype(jnp.bfloat16)
    partials = _sc_partial_sums(idx, table16)
    partials = (
        partials.reshape(NW, HIST, 2, 2, 16)
        .transpose(0, 1, 2, 4, 3)
        .reshape(NW, HIST, D)
    )
    return _project(partials, W, b.reshape(1, VOCAB))


# speedup vs baseline: 1.3087x; 1.3087x over previous
import functools

import jax
import jax.numpy as jnp
from jax import lax
from jax.experimental import pallas as pl
from jax.experimental.pallas import tpu as pltpu
from jax.experimental.pallas import tpu_sc as plsc

VOCAB = 100000
D = 64
BATCH = 16384
HIST = 50

NC = 2
NS = 16
NW = NC * NS

_mesh = plsc.VectorSubcoreMesh(core_axis_name="c", subcore_axis_name="s")


@functools.partial(
    pl.kernel,
    mesh=_mesh,
    out_type=jax.ShapeDtypeStruct((HIST, VOCAB), jnp.float32),
    scratch_types=[
        pltpu.VMEM((BATCH,), jnp.int32),
        pltpu.VMEM((VOCAB,), jnp.float32),
        pltpu.SemaphoreType.DMA,
    ],
    compiler_params=pltpu.CompilerParams(
        use_tc_tiling_on_sc=False, needs_layout_passes=False
    ),
)
def _sc_hist(idxT_hbm, out_hbm, idxv, cnt, sem):
    wid = lax.axis_index("s") * NC + lax.axis_index("c")
    zero = jnp.zeros((16,), jnp.float32)
    ones = jnp.ones((16,), jnp.float32)

    for p in range(2):
        l = p * NW + wid

        @pl.when(l < HIST)
        def _():
            pltpu.sync_copy(idxT_hbm.at[l], idxv)

            def zbody(i, carry):
                cnt[pl.ds(i * 16, 16)] = zero
                return carry

            lax.fori_loop(0, VOCAB // 16, zbody, 0)

            def sbody(i, carry):
                iv = idxv[pl.ds(i * 16, 16)]
                plsc.addupdate_scatter(cnt, [iv], ones)
                return carry

            lax.fori_loop(0, BATCH // 16, sbody, 0)

            pltpu.sync_copy(cnt, out_hbm.at[l])


VK = 8192
NK = VOCAB // VK
TAIL = VOCAB - NK * VK


def _mean_body(cnt_ref, t_ref, ctail_ref, ttail_ref, o_ref):
    k = pl.program_id(0)

    @pl.when(k == 0)
    def _():
        o_ref[...] = lax.dot_general(
            ctail_ref[...], ttail_ref[...], (((1,), (0,)), ((), ())),
            preferred_element_type=jnp.float32,
        )

    o_ref[...] += lax.dot_general(
        cnt_ref[...], t_ref[...], (((1,), (0,)), ((), ())),
        preferred_element_type=jnp.float32,
    )

    @pl.when(k == NK - 1)
    def _():
        o_ref[...] *= 1.0 / BATCH


_mean = pl.pallas_call(
    _mean_body,
    grid=(NK,),
    in_specs=[
        pl.BlockSpec((HIST, VK), lambda k: (0, k)),
        pl.BlockSpec((VK, D), lambda k: (k, 0)),
        pl.BlockSpec((HIST, TAIL), lambda k: (0, 0)),
        pl.BlockSpec((TAIL, D), lambda k: (0, 0)),
    ],
    out_specs=pl.BlockSpec((HIST, D), lambda k: (0, 0)),
    out_shape=jax.ShapeDtypeStruct((HIST, D), jnp.float32),
)


VC = 8192


def _mm_body(mean_ref, w_ref, b_ref, o_ref):
    o_ref[...] = (
        lax.dot_general(
            mean_ref[...], w_ref[...], (((1,), (1,)), ((), ())),
            preferred_element_type=jnp.float32,
        )
        + b_ref[...]
    )


_project = pl.pallas_call(
    _mm_body,
    grid=(pl.cdiv(VOCAB, VC),),
    in_specs=[
        pl.BlockSpec((HIST, D), lambda j: (0, 0)),
        pl.BlockSpec((VC, D), lambda j: (j, 0)),
        pl.BlockSpec((1, VC), lambda j: (0, j)),
    ],
    out_specs=pl.BlockSpec((HIST, VC), lambda j: (0, j)),
    out_shape=jax.ShapeDtypeStruct((HIST, VOCAB), jnp.float32),
)


def kernel(context_idxs, emb_table, W, b):
    idxT = context_idxs.astype(jnp.int32).T
    counts = _sc_hist(idxT)
    mean = _mean(
        counts,
        emb_table,
        counts[:, NK * VK :],
        emb_table[NK * VK :],
    )
    return _project(mean, W, b.reshape(1, VOCAB))
